# baseline (speedup 1.0000x reference)
import functools

import jax
import jax.numpy as jnp
from jax import lax
from jax.experimental import pallas as pl
from jax.experimental.pallas import tpu as pltpu
from jax.experimental.pallas import tpu_sc as plsc

N_NODES = 10000
D = 128
NC = 2
NS = 16
NW = NC * NS
CHUNK = 128
N_PAD = 10240
ROWS_PER_TILE = N_PAD // NS
TABLE_PAD = 10008
NBUF = 2


def _make_sc_scatter(K):
    mesh = plsc.VectorSubcoreMesh(core_axis_name="c", subcore_axis_name="s")

    @functools.partial(
        pl.kernel,
        mesh=mesh,
        out_type=jax.ShapeDtypeStruct((NC, N_PAD, D), jnp.float32),
        scratch_types=[
            pltpu.VMEM((NBUF, 2, CHUNK), jnp.int32),
            pltpu.VMEM((CHUNK, D), jnp.float32),
            pltpu.VMEM((CHUNK, D), jnp.float32),
            pltpu.VMEM_SHARED((N_PAD, D), jnp.float32),
            pltpu.SemaphoreType.DMA,
            pltpu.SemaphoreType.DMA,
            pltpu.SemaphoreType.DMA,
            pltpu.SemaphoreType.DMA,
        ],
    )
    def sc_scatter(h_hbm, idx_hbm, out_hbm, idx_v,
                   gbuf0, gbuf1, agg_sh, isem0, isem1, gsem0, gsem1):
        c = lax.axis_index("c")
        s = lax.axis_index("s")
        w = s * NC + c
        gbufs = (gbuf0, gbuf1)
        isems = (isem0, isem1)
        gsems = (gsem0, gsem1)

        zero16 = jnp.zeros((16,), jnp.float32)

        def zrow(r, carry):
            for cc in range(D // 16):
                gbuf0[r, pl.ds(cc * 16, 16)] = zero16
            return carry

        lax.fori_loop(0, CHUNK, zrow, 0)
        base = s * ROWS_PER_TILE
        for j in range(ROWS_PER_TILE // CHUNK):
            pltpu.sync_copy(gbuf0, agg_sh.at[pl.ds(base + j * CHUNK, CHUNK)])
        plsc.subcore_barrier()

        pltpu.async_copy(idx_hbm.at[w, 0], idx_v.at[0], isems[0])
        pltpu.async_copy(idx_hbm.at[w, 1], idx_v.at[1], isems[1])
        pltpu.make_async_copy(idx_hbm.at[w, 0], idx_v.at[0], isems[0]).wait()
        pltpu.async_copy(h_hbm.at[idx_v.at[0, 1]], gbuf0, gsems[0])

        def body(i, carry):
            k0 = i * NBUF
            for b in range(NBUF):
                k = k0 + b
                b2 = (b + 1) % NBUF
                pltpu.make_async_copy(idx_hbm.at[w, 0], idx_v.at[b2],
                                      isems[b2]).wait()
                pltpu.async_copy(h_hbm.at[idx_v.at[b2, 1]], gbufs[b2],
                                 gsems[b2])
                pltpu.make_async_copy(h_hbm.at[idx_v.at[0, 1]], gbufs[b],
                                      gsems[b]).wait()
                pltpu.sync_copy(gbufs[b], agg_sh.at[idx_v.at[b, 0]], add=True)
                pltpu.async_copy(idx_hbm.at[w, k + 2], idx_v.at[b], isems[b])
            return carry

        lax.fori_loop(0, K // NBUF, body, 0)
        pltpu.make_async_copy(h_hbm.at[idx_v.at[0, 1]], gbufs[K % NBUF],
                              gsems[K % NBUF]).wait()
        b_last = (K + 1) % NBUF
        pltpu.make_async_copy(idx_hbm.at[w, 0], idx_v.at[b_last],
                              isems[b_last]).wait()
        plsc.subcore_barrier()

        for j in range(ROWS_PER_TILE // CHUNK):
            pltpu.sync_copy(agg_sh.at[pl.ds(base + j * CHUNK, CHUNK)], gbuf0)
            pltpu.sync_copy(gbuf0, out_hbm.at[c, pl.ds(base + j * CHUNK, CHUNK)])

    return sc_scatter


def _bn_relu(y, g, b):
    m = jnp.mean(y, axis=0, keepdims=True)
    v = jnp.mean((y - m) ** 2, axis=0, keepdims=True)
    return jnp.maximum((y - m) * lax.rsqrt(v + 1e-5) * g + b, 0.0)


def _dense_layer_body(h_ref, a0_ref, a1_ref, W1_ref, b1_ref, g1_ref, bb1_ref,
                      W2_ref, b2_ref, g2_ref, bb2_ref, out_ref):
    t = h_ref[...] + a0_ref[...] + a1_ref[...]
    y = jnp.dot(t, W1_ref[...], preferred_element_type=jnp.float32) + b1_ref[...]
    y = _bn_relu(y, g1_ref[...], bb1_ref[...])
    z = jnp.dot(y, W2_ref[...], preferred_element_type=jnp.float32) + b2_ref[...]
    out_ref[...] = _bn_relu(z, g2_ref[...], bb2_ref[...])


def _dense_pred_body(h_ref, a0_ref, a1_ref, W1_ref, b1_ref, g1_ref, bb1_ref,
                     W2_ref, b2_ref, g2_ref, bb2_ref,
                     h0_ref, P0_ref, P1_ref, P2_ref, pb_ref, out_ref):
    t = h_ref[...] + a0_ref[...] + a1_ref[...]
    y = jnp.dot(t, W1_ref[...], preferred_element_type=jnp.float32) + b1_ref[...]
    y = _bn_relu(y, g1_ref[...], bb1_ref[...])
    z = jnp.dot(y, W2_ref[...], preferred_element_type=jnp.float32) + b2_ref[...]
    h2 = _bn_relu(z, g2_ref[...], bb2_ref[...])
    out_ref[...] = (jnp.dot(h0_ref[...], P0_ref[...], preferred_element_type=jnp.float32)
                    + jnp.dot(h_ref[...], P1_ref[...], preferred_element_type=jnp.float32)
                    + jnp.dot(h2, P2_ref[...], preferred_element_type=jnp.float32)
                    + pb_ref[...])


_dense_layer = pl.pallas_call(
    _dense_layer_body,
    out_shape=jax.ShapeDtypeStruct((N_NODES, D), jnp.float32),
)

_dense_pred = pl.pallas_call(
    _dense_pred_body,
    out_shape=jax.ShapeDtypeStruct((N_NODES, 32), jnp.float32),
)


def kernel(x, edge_index,
           mlp0_W1, mlp0_b1, mlp0_bn_g, mlp0_bn_b, mlp0_W2, mlp0_b2,
           mlp1_W1, mlp1_b1, mlp1_bn_g, mlp1_bn_b, mlp1_W2, mlp1_b2,
           bn0_g, bn0_b, bn1_g, bn1_b,
           pred0_W, pred0_b, pred1_W, pred1_b, pred2_W, pred2_b):
    row = edge_index[0]
    col = edge_index[1]
    E = row.shape[0]
    K = -(-E // (NW * CHUNK * NBUF)) * NBUF
    E_pad = K * NW * CHUNK
    pad = E_pad - E
    row_p = jnp.concatenate([row, jnp.full((pad,), N_PAD - 1, jnp.int32)]).reshape(NW, K, 1, CHUNK)
    col_p = jnp.concatenate([col, jnp.full((pad,), N_NODES, jnp.int32)]).reshape(NW, K, 1, CHUNK)
    idx_p = jnp.concatenate([row_p, col_p], axis=2)
    dummy = jnp.tile(
        jnp.stack([jnp.full((CHUNK,), N_PAD - 1, jnp.int32),
                   jnp.full((CHUNK,), N_NODES, jnp.int32)])[None, None],
        (NW, NBUF, 1, 1))
    idx_p = jnp.concatenate([idx_p, dummy], axis=1)

    sc_scatter = _make_sc_scatter(K)
    zrows = jnp.zeros((TABLE_PAD - N_NODES, D), jnp.float32)

    def r2(v):
        return v.reshape(1, -1)

    h0 = x
    parts0 = sc_scatter(jnp.concatenate([h0, zrows], axis=0), idx_p)
    h1 = _dense_layer(h0, parts0[0, :N_NODES], parts0[1, :N_NODES],
                      mlp0_W1, r2(mlp0_b1), r2(mlp0_bn_g), r2(mlp0_bn_b),
                      mlp0_W2, r2(mlp0_b2), r2(bn0_g), r2(bn0_b))
    parts1 = sc_scatter(jnp.concatenate([h1, zrows], axis=0), idx_p)
    out = _dense_pred(h1, parts1[0, :N_NODES], parts1[1, :N_NODES],
                      mlp1_W1, r2(mlp1_b1), r2(mlp1_bn_g), r2(mlp1_bn_b),
                      mlp1_W2, r2(mlp1_b2), r2(bn1_g), r2(bn1_b),
                      h0, pred0_W, pred1_W, pred2_W,
                      r2(pred0_b + pred1_b + pred2_b))
    return out


# speedup vs baseline: 1.6552x; 1.6552x over previous
import functools

import jax
import jax.numpy as jnp
from jax import lax
from jax.experimental import pallas as pl
from jax.experimental.pallas import tpu as pltpu
from jax.experimental.pallas import tpu_sc as plsc

N_NODES = 10000
D = 128
NC = 2
NS = 16
NW = NC * NS
CHUNK = 128
N_PAD = 10240
ROWS_PER_TILE = N_PAD // NS


def _make_sc_scatter(K):
    mesh = plsc.VectorSubcoreMesh(core_axis_name="c", subcore_axis_name="s")

    @functools.partial(
        pl.kernel,
        mesh=mesh,
        out_type=jax.ShapeDtypeStruct((NC, N_PAD, D), jnp.float32),
        scratch_types=[
            pltpu.VMEM((K, CHUNK), jnp.int32),
            pltpu.VMEM((K, CHUNK), jnp.int32),
            pltpu.VMEM((CHUNK, D), jnp.float32),
            pltpu.VMEM_SHARED((N_PAD, D), jnp.float32),
            pltpu.SemaphoreType.DMA,
        ],
    )
    def sc_scatter(h_hbm, row_hbm, col_hbm, out_hbm, row_v, col_v, gbuf,
                   agg_sh, sem):
        c = lax.axis_index("c")
        s = lax.axis_index("s")
        w = s * NC + c

        zero16 = jnp.zeros((16,), jnp.float32)

        def zrow(r, carry):
            for cc in range(D // 16):
                gbuf[r, pl.ds(cc * 16, 16)] = zero16
            return carry

        lax.fori_loop(0, CHUNK, zrow, 0)
        base = s * ROWS_PER_TILE
        for j in range(ROWS_PER_TILE // CHUNK):
            pltpu.sync_copy(gbuf, agg_sh.at[pl.ds(base + j * CHUNK, CHUNK)])
        plsc.subcore_barrier()

        pltpu.sync_copy(row_hbm.at[w], row_v)
        pltpu.sync_copy(col_hbm.at[w], col_v)

        def body(k, carry):
            pltpu.async_copy(h_hbm.at[col_v.at[k]], gbuf, sem).wait()
            pltpu.sync_copy(gbuf, agg_sh.at[row_v.at[k]], add=True)
            return carry

        lax.fori_loop(0, K, body, 0)
        plsc.subcore_barrier()

        for j in range(ROWS_PER_TILE // CHUNK):
            pltpu.sync_copy(agg_sh.at[pl.ds(base + j * CHUNK, CHUNK)], gbuf)
            pltpu.sync_copy(gbuf, out_hbm.at[c, pl.ds(base + j * CHUNK, CHUNK)])

    return sc_scatter


def _bn_relu(y, g, b):
    m = jnp.mean(y, axis=0, keepdims=True)
    v = jnp.mean((y - m) ** 2, axis=0, keepdims=True)
    return jnp.maximum((y - m) * lax.rsqrt(v + 1e-5) * g + b, 0.0)


def _dense_layer_body(h_ref, a0_ref, a1_ref, W1_ref, b1_ref, g1_ref, bb1_ref,
                      W2_ref, b2_ref, g2_ref, bb2_ref, out_ref):
    t = h_ref[...] + a0_ref[...] + a1_ref[...]
    y = jnp.dot(t, W1_ref[...], preferred_element_type=jnp.float32) + b1_ref[...]
    y = _bn_relu(y, g1_ref[...], bb1_ref[...])
    z = jnp.dot(y, W2_ref[...], preferred_element_type=jnp.float32) + b2_ref[...]
    out_ref[...] = _bn_relu(z, g2_ref[...], bb2_ref[...])


def _dense_pred_body(h_ref, a0_ref, a1_ref, W1_ref, b1_ref, g1_ref, bb1_ref,
                     W2_ref, b2_ref, g2_ref, bb2_ref,
                     h0_ref, P0_ref, P1_ref, P2_ref, pb_ref, out_ref):
    t = h_ref[...] + a0_ref[...] + a1_ref[...]
    y = jnp.dot(t, W1_ref[...], preferred_element_type=jnp.float32) + b1_ref[...]
    y = _bn_relu(y, g1_ref[...], bb1_ref[...])
    z = jnp.dot(y, W2_ref[...], preferred_element_type=jnp.float32) + b2_ref[...]
    h2 = _bn_relu(z, g2_ref[...], bb2_ref[...])
    out_ref[...] = (jnp.dot(h0_ref[...], P0_ref[...], preferred_element_type=jnp.float32)
                    + jnp.dot(h_ref[...], P1_ref[...], preferred_element_type=jnp.float32)
                    + jnp.dot(h2, P2_ref[...], preferred_element_type=jnp.float32)
                    + pb_ref[...])


_dense_layer = pl.pallas_call(
    _dense_layer_body,
    out_shape=jax.ShapeDtypeStruct((N_NODES, D), jnp.float32),
)

_dense_pred = pl.pallas_call(
    _dense_pred_body,
    out_shape=jax.ShapeDtypeStruct((N_NODES, 32), jnp.float32),
)


def kernel(x, edge_index,
           mlp0_W1, mlp0_b1, mlp0_bn_g, mlp0_bn_b, mlp0_W2, mlp0_b2,
           mlp1_W1, mlp1_b1, mlp1_bn_g, mlp1_bn_b, mlp1_W2, mlp1_b2,
           bn0_g, bn0_b, bn1_g, bn1_b,
           pred0_W, pred0_b, pred1_W, pred1_b, pred2_W, pred2_b):
    row = edge_index[0]
    col = edge_index[1]
    E = row.shape[0]
    K = -(-E // (NW * CHUNK))
    E_pad = K * NW * CHUNK
    pad = E_pad - E
    row_p = jnp.concatenate([row, jnp.full((pad,), N_PAD - 1, jnp.int32)]).reshape(NW, K, CHUNK)
    col_p = jnp.concatenate([col, jnp.zeros((pad,), jnp.int32)]).reshape(NW, K, CHUNK)

    sc_scatter = _make_sc_scatter(K)

    def r2(v):
        return v.reshape(1, -1)

    h0 = x
    parts0 = sc_scatter(h0, row_p, col_p)
    h1 = _dense_layer(h0, parts0[0, :N_NODES], parts0[1, :N_NODES],
                      mlp0_W1, r2(mlp0_b1), r2(mlp0_bn_g), r2(mlp0_bn_b),
                      mlp0_W2, r2(mlp0_b2), r2(bn0_g), r2(bn0_b))
    parts1 = sc_scatter(h1, row_p, col_p)
    out = _dense_pred(h1, parts1[0, :N_NODES], parts1[1, :N_NODES],
                      mlp1_W1, r2(mlp1_b1), r2(mlp1_bn_g), r2(mlp1_bn_b),
                      mlp1_W2, r2(mlp1_b2), r2(bn1_g), r2(bn1_b),
                      h0, pred0_W, pred1_W, pred2_W,
                      r2(pred0_b + pred1_b + pred2_b))
    return out


# speedup vs baseline: 1.8461x; 1.1154x over previous
import functools

import jax
import jax.numpy as jnp
from jax import lax
from jax.experimental import pallas as pl
from jax.experimental.pallas import tpu as pltpu
from jax.experimental.pallas import tpu_sc as plsc

N_NODES = 10000
D = 128
NC = 2
NS = 16
NW = NC * NS
CHUNK = 128
N_PAD = 10240
ROWS_PER_TILE = N_PAD // NS


def _make_sc_scatter(K0, K1):
    mesh = plsc.VectorSubcoreMesh(core_axis_name="c", subcore_axis_name="s")
    KMAX = max(K0, K1)

    @functools.partial(
        pl.kernel,
        mesh=mesh,
        out_type=jax.ShapeDtypeStruct((NC, N_PAD, D), jnp.float32),
        scratch_types=[
            pltpu.VMEM((2, KMAX, CHUNK), jnp.int32),
            pltpu.VMEM((CHUNK, D), jnp.float32),
            pltpu.VMEM_SHARED((N_PAD, D), jnp.float32),
            pltpu.SemaphoreType.DMA,
        ],
    )
    def sc_scatter(h_hbm, idx_hbm, out_hbm, idx_v, gbuf, agg_sh, sem):
        c = lax.axis_index("c")
        s = lax.axis_index("s")
        kc = jnp.where(c == 0, K0, K1)

        zero16 = jnp.zeros((16,), jnp.float32)

        def zrow(r, carry):
            for cc in range(D // 16):
                gbuf[r, pl.ds(cc * 16, 16)] = zero16
            return carry

        lax.fori_loop(0, CHUNK, zrow, 0)
        base = s * ROWS_PER_TILE
        for j in range(ROWS_PER_TILE // CHUNK):
            pltpu.sync_copy(gbuf, agg_sh.at[pl.ds(base + j * CHUNK, CHUNK)])
        plsc.subcore_barrier()

        pltpu.sync_copy(idx_hbm.at[c, s], idx_v)

        def body(k, carry):
            pltpu.async_copy(h_hbm.at[idx_v.at[1, k]], gbuf, sem).wait()
            pltpu.sync_copy(gbuf, agg_sh.at[idx_v.at[0, k]], add=True)
            return carry

        lax.fori_loop(0, kc, body, 0)
        plsc.subcore_barrier()

        for j in range(ROWS_PER_TILE // CHUNK):
            pltpu.sync_copy(agg_sh.at[pl.ds(base + j * CHUNK, CHUNK)], gbuf)
            pltpu.sync_copy(gbuf, out_hbm.at[c, pl.ds(base + j * CHUNK, CHUNK)])

    return sc_scatter


def _bn_relu(y, g, b):
    m = jnp.mean(y, axis=0, keepdims=True)
    v = jnp.mean((y - m) ** 2, axis=0, keepdims=True)
    return jnp.maximum((y - m) * lax.rsqrt(v + 1e-5) * g + b, 0.0)


def _dense_layer_body(h_ref, a0_ref, a1_ref, W1_ref, b1_ref, g1_ref, bb1_ref,
                      W2_ref, b2_ref, g2_ref, bb2_ref, out_ref):
    t = h_ref[...] + a0_ref[...] + a1_ref[...]
    y = jnp.dot(t, W1_ref[...], preferred_element_type=jnp.float32) + b1_ref[...]
    y = _bn_relu(y, g1_ref[...], bb1_ref[...])
    z = jnp.dot(y, W2_ref[...], preferred_element_type=jnp.float32) + b2_ref[...]
    out_ref[...] = _bn_relu(z, g2_ref[...], bb2_ref[...])


def _dense_pred_body(h_ref, a0_ref, a1_ref, W1_ref, b1_ref, g1_ref, bb1_ref,
                     W2_ref, b2_ref, g2_ref, bb2_ref,
                     h0_ref, P0_ref, P1_ref, P2_ref, pb_ref, out_ref):
    t = h_ref[...] + a0_ref[...] + a1_ref[...]
    y = jnp.dot(t, W1_ref[...], preferred_element_type=jnp.float32) + b1_ref[...]
    y = _bn_relu(y, g1_ref[...], bb1_ref[...])
    z = jnp.dot(y, W2_ref[...], preferred_element_type=jnp.float32) + b2_ref[...]
    h2 = _bn_relu(z, g2_ref[...], bb2_ref[...])
    out_ref[...] = (jnp.dot(h0_ref[...], P0_ref[...], preferred_element_type=jnp.float32)
                    + jnp.dot(h_ref[...], P1_ref[...], preferred_element_type=jnp.float32)
                    + jnp.dot(h2, P2_ref[...], preferred_element_type=jnp.float32)
                    + pb_ref[...])


_dense_layer = pl.pallas_call(
    _dense_layer_body,
    out_shape=jax.ShapeDtypeStruct((N_NODES, D), jnp.float32),
)

_dense_pred = pl.pallas_call(
    _dense_pred_body,
    out_shape=jax.ShapeDtypeStruct((N_NODES, 32), jnp.float32),
)


def kernel(x, edge_index,
           mlp0_W1, mlp0_b1, mlp0_bn_g, mlp0_bn_b, mlp0_W2, mlp0_b2,
           mlp1_W1, mlp1_b1, mlp1_bn_g, mlp1_bn_b, mlp1_W2, mlp1_b2,
           bn0_g, bn0_b, bn1_g, bn1_b,
           pred0_W, pred0_b, pred1_W, pred1_b, pred2_W, pred2_b):
    row = edge_index[0]
    col = edge_index[1]
    E = row.shape[0]
    F0 = 0.35
    T = -(-E // CHUNK)
    K0 = max(1, round(T * F0 / NS))
    K1 = -(-(T - NS * K0) // NS)
    KMAX = max(K0, K1)
    E_pad = NS * (K0 + K1) * CHUNK
    pad = E_pad - E
    rowf = jnp.concatenate([row, jnp.full((pad,), N_PAD - 1, jnp.int32)])
    colf = jnp.concatenate([col, jnp.zeros((pad,), jnp.int32)])
    split = NS * K0 * CHUNK

    def _part(v, start, kcount):
        a = v[start:start + NS * kcount * CHUNK].reshape(NS, kcount, CHUNK)
        if kcount < KMAX:
            a = jnp.concatenate(
                [a, jnp.zeros((NS, KMAX - kcount, CHUNK), jnp.int32)], axis=1)
        return a

    idx_p = jnp.stack([
        jnp.stack([_part(rowf, 0, K0), _part(colf, 0, K0)], axis=1),
        jnp.stack([_part(rowf, split, K1), _part(colf, split, K1)], axis=1),
    ])

    sc_scatter = _make_sc_scatter(K0, K1)

    def r2(v):
        return v.reshape(1, -1)

    h0 = x
    parts0 = sc_scatter(h0, idx_p)
    h1 = _dense_layer(h0, parts0[0, :N_NODES], parts0[1, :N_NODES],
                      mlp0_W1, r2(mlp0_b1), r2(mlp0_bn_g), r2(mlp0_bn_b),
                      mlp0_W2, r2(mlp0_b2), r2(bn0_g), r2(bn0_b))
    parts1 = sc_scatter(h1, idx_p)
    out = _dense_pred(h1, parts1[0, :N_NODES], parts1[1, :N_NODES],
                      mlp1_W1, r2(mlp1_b1), r2(mlp1_bn_g), r2(mlp1_bn_b),
                      mlp1_W2, r2(mlp1_b2), r2(bn1_g), r2(bn1_b),
                      h0, pred0_W, pred1_W, pred2_W,
                      r2(pred0_b + pred1_b + pred2_b))
    return out


# speedup vs baseline: 1.9580x; 1.0606x over previous
import functools

import jax
import jax.numpy as jnp
from jax import lax
from jax.experimental import pallas as pl
from jax.experimental.pallas import tpu as pltpu
from jax.experimental.pallas import tpu_sc as plsc

N_NODES = 10000
D = 128
NC = 2
NS = 16
NW = NC * NS
CHUNK = 128
N_PAD = 10240
ROWS_PER_TILE = N_PAD // NS


def _make_sc_scatter(K0, K1):
    mesh = plsc.VectorSubcoreMesh(core_axis_name="c", subcore_axis_name="s")
    KMAX = max(K0, K1)

    @functools.partial(
        pl.kernel,
        mesh=mesh,
        out_type=jax.ShapeDtypeStruct((NC, N_PAD, D), jnp.float32),
        scratch_types=[
            pltpu.VMEM((2, KMAX, CHUNK), jnp.int32),
            pltpu.VMEM((CHUNK, D), jnp.float32),
            pltpu.VMEM_SHARED((N_PAD, D), jnp.float32),
            pltpu.SemaphoreType.DMA,
        ],
    )
    def sc_scatter(h_hbm, idx_hbm, out_hbm, idx_v, gbuf, agg_sh, sem):
        c = lax.axis_index("c")
        s = lax.axis_index("s")
        kc = jnp.where(c == 0, K0, K1)

        zero16 = jnp.zeros((16,), jnp.float32)

        def zrow(r, carry):
            for cc in range(D // 16):
                gbuf[r, pl.ds(cc * 16, 16)] = zero16
            return carry

        lax.fori_loop(0, CHUNK, zrow, 0)
        base = s * ROWS_PER_TILE
        for j in range(ROWS_PER_TILE // CHUNK):
            pltpu.sync_copy(gbuf, agg_sh.at[pl.ds(base + j * CHUNK, CHUNK)])
        plsc.subcore_barrier()

        pltpu.sync_copy(idx_hbm.at[c, s], idx_v)

        def body(k, carry):
            pltpu.async_copy(h_hbm.at[idx_v.at[1, k]], gbuf, sem).wait()
            pltpu.sync_copy(gbuf, agg_sh.at[idx_v.at[0, k]], add=True)
            return carry

        lax.fori_loop(0, kc, body, 0)
        plsc.subcore_barrier()

        for j in range(ROWS_PER_TILE // CHUNK):
            pltpu.sync_copy(agg_sh.at[pl.ds(base + j * CHUNK, CHUNK)], gbuf)
            pltpu.sync_copy(gbuf, out_hbm.at[c, pl.ds(base + j * CHUNK, CHUNK)])

    return sc_scatter


def _bn_relu(y, g, b):
    m = jnp.mean(y, axis=0, keepdims=True)
    v = jnp.mean((y - m) ** 2, axis=0, keepdims=True)
    return jnp.maximum((y - m) * lax.rsqrt(v + 1e-5) * g + b, 0.0)


def _dense_layer_body(h_ref, a0_ref, a1_ref, W1_ref, b1_ref, g1_ref, bb1_ref,
                      W2_ref, b2_ref, g2_ref, bb2_ref, out_ref):
    t = h_ref[...] + a0_ref[...] + a1_ref[...]
    y = jnp.dot(t, W1_ref[...], preferred_element_type=jnp.float32) + b1_ref[...]
    y = _bn_relu(y, g1_ref[...], bb1_ref[...])
    z = jnp.dot(y, W2_ref[...], preferred_element_type=jnp.float32) + b2_ref[...]
    out_ref[...] = _bn_relu(z, g2_ref[...], bb2_ref[...])


def _dense_pred_body(h_ref, a0_ref, a1_ref, W1_ref, b1_ref, g1_ref, bb1_ref,
                     W2_ref, b2_ref, g2_ref, bb2_ref,
                     h0_ref, P0_ref, P1_ref, P2_ref, pb_ref, out_ref):
    t = h_ref[...] + a0_ref[...] + a1_ref[...]
    y = jnp.dot(t, W1_ref[...], preferred_element_type=jnp.float32) + b1_ref[...]
    y = _bn_relu(y, g1_ref[...], bb1_ref[...])
    z = jnp.dot(y, W2_ref[...], preferred_element_type=jnp.float32) + b2_ref[...]
    h2 = _bn_relu(z, g2_ref[...], bb2_ref[...])
    out_ref[...] = (jnp.dot(h0_ref[...], P0_ref[...], preferred_element_type=jnp.float32)
                    + jnp.dot(h_ref[...], P1_ref[...], preferred_element_type=jnp.float32)
                    + jnp.dot(h2, P2_ref[...], preferred_element_type=jnp.float32)
                    + pb_ref[...])


_dense_layer = pl.pallas_call(
    _dense_layer_body,
    out_shape=jax.ShapeDtypeStruct((N_NODES, D), jnp.float32),
)

_dense_pred = pl.pallas_call(
    _dense_pred_body,
    out_shape=jax.ShapeDtypeStruct((N_NODES, 32), jnp.float32),
)


def kernel(x, edge_index,
           mlp0_W1, mlp0_b1, mlp0_bn_g, mlp0_bn_b, mlp0_W2, mlp0_b2,
           mlp1_W1, mlp1_b1, mlp1_bn_g, mlp1_bn_b, mlp1_W2, mlp1_b2,
           bn0_g, bn0_b, bn1_g, bn1_b,
           pred0_W, pred0_b, pred1_W, pred1_b, pred2_W, pred2_b):
    row = edge_index[0]
    col = edge_index[1]
    E = row.shape[0]
    F0 = 0.45
    T = -(-E // CHUNK)
    K0 = max(1, round(T * F0 / NS))
    K1 = -(-(T - NS * K0) // NS)
    KMAX = max(K0, K1)
    E_pad = NS * (K0 + K1) * CHUNK
    pad = E_pad - E
    rowf = jnp.concatenate([row, jnp.full((pad,), N_PAD - 1, jnp.int32)])
    colf = jnp.concatenate([col, jnp.zeros((pad,), jnp.int32)])
    split = NS * K0 * CHUNK

    def _part(v, start, kcount):
        a = v[start:start + NS * kcount * CHUNK].reshape(NS, kcount, CHUNK)
        if kcount < KMAX:
            a = jnp.concatenate(
                [a, jnp.zeros((NS, KMAX - kcount, CHUNK), jnp.int32)], axis=1)
        return a

    idx_p = jnp.stack([
        jnp.stack([_part(rowf, 0, K0), _part(colf, 0, K0)], axis=1),
        jnp.stack([_part(rowf, split, K1), _part(colf, split, K1)], axis=1),
    ])

    sc_scatter = _make_sc_scatter(K0, K1)

    def r2(v):
        return v.reshape(1, -1)

    h0 = x
    parts0 = sc_scatter(h0, idx_p)
    h1 = _dense_layer(h0, parts0[0, :N_NODES], parts0[1, :N_NODES],
                      mlp0_W1, r2(mlp0_b1), r2(mlp0_bn_g), r2(mlp0_bn_b),
                      mlp0_W2, r2(mlp0_b2), r2(bn0_g), r2(bn0_b))
    parts1 = sc_scatter(h1, idx_p)
    out = _dense_pred(h1, parts1[0, :N_NODES], parts1[1, :N_NODES],
                      mlp1_W1, r2(mlp1_b1), r2(mlp1_bn_g), r2(mlp1_bn_b),
                      mlp1_W2, r2(mlp1_b2), r2(bn1_g), r2(bn1_b),
                      h0, pred0_W, pred1_W, pred2_W,
                      r2(pred0_b + pred1_b + pred2_b))
    return out


# speedup vs baseline: 2.3313x; 1.1906x over previous
import functools

import jax
import jax.numpy as jnp
from jax import lax
from jax.experimental import pallas as pl
from jax.experimental.pallas import tpu as pltpu
from jax.experimental.pallas import tpu_sc as plsc

N_NODES = 10000
D = 128
NC = 2
NS = 16
NW = NC * NS
CHUNK = 128
N_PAD = 10240
ROWS_PER_TILE = N_PAD // NS


def _make_sc_scatter(K0, K1):
    mesh = plsc.VectorSubcoreMesh(core_axis_name="c", subcore_axis_name="s")
    KMAX = max(K0, K1)

    @functools.partial(
        pl.kernel,
        mesh=mesh,
        out_type=jax.ShapeDtypeStruct((NC, N_PAD, D), jnp.float32),
        scratch_types=[
            pltpu.VMEM((2, KMAX, CHUNK), jnp.int32),
            pltpu.VMEM((CHUNK, D), jnp.float32),
            pltpu.VMEM_SHARED((N_PAD, D), jnp.float32),
            pltpu.SemaphoreType.DMA,
        ],
    )
    def sc_scatter(h_hbm, idx_hbm, out_hbm, idx_v, gbuf, agg_sh, sem):
        c = lax.axis_index("c")
        s = lax.axis_index("s")
        kc = jnp.where(c == 0, K0, K1)

        zero16 = jnp.zeros((16,), jnp.float32)

        def zrow(r, carry):
            for cc in range(D // 16):
                gbuf[r, pl.ds(cc * 16, 16)] = zero16
            return carry

        lax.fori_loop(0, CHUNK, zrow, 0)
        base = s * ROWS_PER_TILE
        for j in range(ROWS_PER_TILE // CHUNK):
            pltpu.sync_copy(gbuf, agg_sh.at[pl.ds(base + j * CHUNK, CHUNK)])
        plsc.subcore_barrier()

        pltpu.sync_copy(idx_hbm.at[c, s], idx_v)

        def body(k, carry):
            pltpu.async_copy(h_hbm.at[idx_v.at[1, k]], gbuf, sem).wait()
            pltpu.sync_copy(gbuf, agg_sh.at[idx_v.at[0, k]], add=True)
            return carry

        lax.fori_loop(0, kc, body, 0)
        plsc.subcore_barrier()

        for j in range(ROWS_PER_TILE // CHUNK):
            pltpu.sync_copy(agg_sh.at[pl.ds(base + j * CHUNK, CHUNK)], gbuf)
            pltpu.sync_copy(gbuf, out_hbm.at[c, pl.ds(base + j * CHUNK, CHUNK)])

    return sc_scatter


def _bn_relu(y, g, b):
    m = jnp.mean(y, axis=0, keepdims=True)
    v = jnp.mean((y - m) ** 2, axis=0, keepdims=True)
    return jnp.maximum((y - m) * lax.rsqrt(v + 1e-5) * g + b, 0.0)


def _dense_layer_body(h_ref, a0_ref, a1_ref, W1_ref, b1_ref, g1_ref, bb1_ref,
                      W2_ref, b2_ref, g2_ref, bb2_ref, out_ref):
    t = h_ref[...] + a0_ref[...] + a1_ref[...]
    y = jnp.dot(t, W1_ref[...], preferred_element_type=jnp.float32) + b1_ref[...]
    y = _bn_relu(y, g1_ref[...], bb1_ref[...])
    z = jnp.dot(y, W2_ref[...], preferred_element_type=jnp.float32) + b2_ref[...]
    out_ref[...] = _bn_relu(z, g2_ref[...], bb2_ref[...])


def _dense_pred_body(h_ref, a0_ref, a1_ref, W1_ref, b1_ref, g1_ref, bb1_ref,
                     W2_ref, b2_ref, g2_ref, bb2_ref,
                     h0_ref, P0_ref, P1_ref, P2_ref, pb_ref, out_ref):
    t = h_ref[...] + a0_ref[...] + a1_ref[...]
    y = jnp.dot(t, W1_ref[...], preferred_element_type=jnp.float32) + b1_ref[...]
    y = _bn_relu(y, g1_ref[...], bb1_ref[...])
    z = jnp.dot(y, W2_ref[...], preferred_element_type=jnp.float32) + b2_ref[...]
    h2 = _bn_relu(z, g2_ref[...], bb2_ref[...])
    out_ref[...] = (jnp.dot(h0_ref[...], P0_ref[...], preferred_element_type=jnp.float32)
                    + jnp.dot(h_ref[...], P1_ref[...], preferred_element_type=jnp.float32)
                    + jnp.dot(h2, P2_ref[...], preferred_element_type=jnp.float32)
                    + pb_ref[...])


_dense_layer = pl.pallas_call(
    _dense_layer_body,
    out_shape=jax.ShapeDtypeStruct((N_NODES, D), jnp.float32),
)

_dense_pred = pl.pallas_call(
    _dense_pred_body,
    out_shape=jax.ShapeDtypeStruct((N_NODES, 32), jnp.float32),
)


def kernel(x, edge_index,
           mlp0_W1, mlp0_b1, mlp0_bn_g, mlp0_bn_b, mlp0_W2, mlp0_b2,
           mlp1_W1, mlp1_b1, mlp1_bn_g, mlp1_bn_b, mlp1_W2, mlp1_b2,
           bn0_g, bn0_b, bn1_g, bn1_b,
           pred0_W, pred0_b, pred1_W, pred1_b, pred2_W, pred2_b):
    row = edge_index[0]
    col = edge_index[1]
    E = row.shape[0]
    F0 = 0.57
    T = -(-E // CHUNK)
    K0 = max(1, round(T * F0 / NS))
    K1 = -(-(T - NS * K0) // NS)
    KMAX = max(K0, K1)
    E_pad = NS * (K0 + K1) * CHUNK
    pad = E_pad - E
    rowf = jnp.concatenate([row, jnp.full((pad,), N_PAD - 1, jnp.int32)])
    colf = jnp.concatenate([col, jnp.zeros((pad,), jnp.int32)])
    split = NS * K0 * CHUNK

    def _part(v, start, kcount):
        a = v[start:start + NS * kcount * CHUNK].reshape(NS, kcount, CHUNK)
        if kcount < KMAX:
            a = jnp.concatenate(
                [a, jnp.zeros((NS, KMAX - kcount, CHUNK), jnp.int32)], axis=1)
        return a

    idx_p = jnp.stack([
        jnp.stack([_part(rowf, 0, K0), _part(colf, 0, K0)], axis=1),
        jnp.stack([_part(rowf, split, K1), _part(colf, split, K1)], axis=1),
    ])

    sc_scatter = _make_sc_scatter(K0, K1)

    def r2(v):
        return v.reshape(1, -1)

    h0 = x
    parts0 = sc_scatter(h0, idx_p)
    h1 = _dense_layer(h0, parts0[0, :N_NODES], parts0[1, :N_NODES],
                      mlp0_W1, r2(mlp0_b1), r2(mlp0_bn_g), r2(mlp0_bn_b),
                      mlp0_W2, r2(mlp0_b2), r2(bn0_g), r2(bn0_b))
    parts1 = sc_scatter(h1, idx_p)
    out = _dense_pred(h1, parts1[0, :N_NODES], parts1[1, :N_NODES],
                      mlp1_W1, r2(mlp1_b1), r2(mlp1_bn_g), r2(mlp1_bn_b),
                      mlp1_W2, r2(mlp1_b2), r2(bn1_g), r2(bn1_b),
                      h0, pred0_W, pred1_W, pred2_W,
                      r2(pred0_b + pred1_b + pred2_b))
    return out
